# baseline (speedup 1.0000x reference)
import functools

import jax
import jax.numpy as jnp
from jax import lax
from jax.experimental import pallas as pl
from jax.experimental.pallas import tpu as pltpu
from jax.experimental.pallas import tpu_sc as plsc

_BATCH = 16384
_HIST = 200
_D = 64
_HB = 256
_GS = 128


@functools.cache
def _build():
    info = plsc.get_sparse_core_info()
    nc, ns = info.num_cores, info.num_subcores
    nw = nc * ns
    nb = _BATCH // nw
    nhalf = nb // _HB

    mesh = plsc.VectorSubcoreMesh(core_axis_name="c", subcore_axis_name="s")

    def body(xt_hbm, tab_hbm, out_hbm, idx0, idx1, rows0, rows1, st0, st1,
             si0, si1, sg0, sg1, so0, so1):
        idx = (idx0, idx1)
        rows = (rows0, rows1)
        stage = (st0, st1)
        si = (si0, si1)
        sg = (sg0, sg1)
        so = (so0, so1)
        wid = lax.axis_index("s") * nc + lax.axis_index("c")
        b0 = wid * nb
        bt0 = wid * (nb // 128)
        iota16 = lax.iota(jnp.int32, 16)

        def fire_idx(h, half):
            b = half
            pltpu.async_copy(
                xt_hbm.at[h, pl.ds(b0 + half * _HB, _HB)], idx[b], si[b])

        def transpose_prev(bp, drain=True):
            if drain:
                pltpu.make_async_copy(
                    stage[bp], out_hbm.at[0, :, pl.ds(0, 2), :, :],
                    so[bp]).wait()

            @pl.loop(0, 16)
            def _(rblk):
                rvec = rblk * 16 + iota16
                btl = jnp.full((16,), rblk // 8, jnp.int32)
                bi = (rblk % 8) * 16 + iota16
                for d in range(_D):
                    v = plsc.load_gather(
                        rows[bp], [rvec, jnp.full((16,), d, jnp.int32)])
                    plsc.store_scatter(
                        stage[bp],
                        [jnp.full((16,), d // 8, jnp.int32), btl,
                         jnp.full((16,), d % 8, jnp.int32), bi],
                        v)

        def fire_store(h, half):
            bp = half
            pltpu.async_copy(
                stage[bp],
                out_hbm.at[h, :, pl.ds(bt0 + half * 2, 2), :, :],
                so[bp])

        def step(h, half, transpose_h, prefetch, drain=True):
            b = half
            pltpu.make_async_copy(
                xt_hbm.at[0, pl.ds(0, _HB)], idx[b], si[b]).wait()
            descs = [
                pltpu.async_copy(
                    tab_hbm.at[idx[b].at[pl.ds(j * _GS, _GS)]],
                    rows[b].at[pl.ds(j * _GS, _GS), :],
                    sg[b],
                )
                for j in range(_HB // _GS)
            ]
            if transpose_h is not None:
                ph, phalf = transpose_h
                transpose_prev(phalf, drain=drain)
                fire_store(ph, phalf)
            for d in descs:
                d.wait()
            if prefetch:
                fire_idx(h + 1, half)

        fire_idx(0, 0)
        fire_idx(0, 1)
        step(0, 0, None, prefetch=True)
        step(0, 1, (0, 0), prefetch=True, drain=False)
        step(1, 0, (0, 1), prefetch=True, drain=False)
        step(1, 1, (1, 0), prefetch=True)

        @pl.loop(2, _HIST - 1)
        def _(h):
            step(h, 0, (h - 1, 1), prefetch=True)
            step(h, 1, (h, 0), prefetch=True)

        h = _HIST - 1
        step(h, 0, (h - 1, 1), prefetch=False)
        step(h, 1, (h, 0), prefetch=False)
        transpose_prev(1)
        fire_store(h, 1)
        pltpu.make_async_copy(
            stage[0], out_hbm.at[0, :, pl.ds(0, 2), :, :], so[0]).wait()
        pltpu.make_async_copy(
            stage[1], out_hbm.at[0, :, pl.ds(0, 2), :, :], so[1]).wait()

    run = pl.kernel(
        body,
        out_type=jax.ShapeDtypeStruct(
            (_HIST, _D // 8, _BATCH // 128, 8, 128), jnp.float32),
        mesh=mesh,
        scratch_types=[
            pltpu.VMEM((_HB,), jnp.int32),
            pltpu.VMEM((_HB,), jnp.int32),
            pltpu.VMEM((_HB, _D), jnp.float32),
            pltpu.VMEM((_HB, _D), jnp.float32),
            pltpu.VMEM((_D // 8, 2, 8, 128), jnp.float32),
            pltpu.VMEM((_D // 8, 2, 8, 128), jnp.float32),
            pltpu.SemaphoreType.DMA,
            pltpu.SemaphoreType.DMA,
            pltpu.SemaphoreType.DMA,
            pltpu.SemaphoreType.DMA,
            pltpu.SemaphoreType.DMA,
            pltpu.SemaphoreType.DMA,
        ],
        compiler_params=pltpu.CompilerParams(
            use_tc_tiling_on_sc=False, needs_layout_passes=False),
    )
    return run


def kernel(x, embeddings):
    run = _build()
    out5 = run(x.T.astype(jnp.int32), embeddings)
    return out5.transpose(2, 4, 0, 1, 3).reshape(_BATCH, _HIST, _D)


# speedup vs baseline: 1.7213x; 1.7213x over previous
import functools

import jax
import jax.numpy as jnp
from jax import lax
from jax.experimental import pallas as pl
from jax.experimental.pallas import tpu as pltpu
from jax.experimental.pallas import tpu_sc as plsc

_BATCH = 16384
_HIST = 200
_D = 64
_HB = 256
_GS = 128


@functools.cache
def _build():
    info = plsc.get_sparse_core_info()
    nc, ns = info.num_cores, info.num_subcores
    nw = nc * ns
    nb = _BATCH // nw
    nhalf = nb // _HB

    mesh = plsc.VectorSubcoreMesh(core_axis_name="c", subcore_axis_name="s")

    def body(xt_hbm, tab_hbm, out_hbm, idx0, idx1, rows0, rows1, st0, st1,
             si0, si1, sg0, sg1, so0, so1):
        idx = (idx0, idx1)
        rows = (rows0, rows1)
        stage = (st0, st1)
        si = (si0, si1)
        sg = (sg0, sg1)
        so = (so0, so1)
        wid = lax.axis_index("s") * nc + lax.axis_index("c")
        b0 = wid * nb
        bt0 = wid * (nb // 128)
        iota16 = lax.iota(jnp.int32, 16)

        def fire_idx(h, half):
            b = half
            pltpu.async_copy(
                xt_hbm.at[h, pl.ds(b0 + half * _HB, _HB)], idx[b], si[b])

        def transpose_prev(bp, drain=True):
            if drain:
                pltpu.make_async_copy(
                    stage[bp], out_hbm.at[0, :, pl.ds(0, 2), :, :],
                    so[bp]).wait()

            @pl.loop(0, 16)
            def _(rblk):
                rvec = rblk * 16 + iota16
                btl = rblk // 8
                bi0 = (rblk % 8) * 16
                dvec0 = jnp.zeros((16,), jnp.int32)

                @plsc.parallel_loop(0, _D, unroll=8, carry=dvec0)
                def _(d, dvec):
                    v = plsc.load_gather(rows[bp], [rvec, dvec])
                    stage[bp][d // 8, btl, d % 8, pl.ds(bi0, 16)] = v
                    return dvec + 1

        def fire_store(h, half):
            bp = half
            pltpu.async_copy(
                stage[bp],
                out_hbm.at[h, :, pl.ds(bt0 + half * 2, 2), :, :],
                so[bp])

        def step(h, half, transpose_h, prefetch, drain=True):
            b = half
            pltpu.make_async_copy(
                xt_hbm.at[0, pl.ds(0, _HB)], idx[b], si[b]).wait()
            descs = [
                pltpu.async_copy(
                    tab_hbm.at[idx[b].at[pl.ds(j * _GS, _GS)]],
                    rows[b].at[pl.ds(j * _GS, _GS), :],
                    sg[b],
                )
                for j in range(_HB // _GS)
            ]
            if transpose_h is not None:
                ph, phalf = transpose_h
                transpose_prev(phalf, drain=drain)
                fire_store(ph, phalf)
            for d in descs:
                d.wait()
            if prefetch:
                fire_idx(h + 1, half)

        fire_idx(0, 0)
        fire_idx(0, 1)
        step(0, 0, None, prefetch=True)
        step(0, 1, (0, 0), prefetch=True, drain=False)
        step(1, 0, (0, 1), prefetch=True, drain=False)
        step(1, 1, (1, 0), prefetch=True)

        @pl.loop(2, _HIST - 1)
        def _(h):
            step(h, 0, (h - 1, 1), prefetch=True)
            step(h, 1, (h, 0), prefetch=True)

        h = _HIST - 1
        step(h, 0, (h - 1, 1), prefetch=False)
        step(h, 1, (h, 0), prefetch=False)
        transpose_prev(1)
        fire_store(h, 1)
        pltpu.make_async_copy(
            stage[0], out_hbm.at[0, :, pl.ds(0, 2), :, :], so[0]).wait()
        pltpu.make_async_copy(
            stage[1], out_hbm.at[0, :, pl.ds(0, 2), :, :], so[1]).wait()

    run = pl.kernel(
        body,
        out_type=jax.ShapeDtypeStruct(
            (_HIST, _D // 8, _BATCH // 128, 8, 128), jnp.float32),
        mesh=mesh,
        scratch_types=[
            pltpu.VMEM((_HB,), jnp.int32),
            pltpu.VMEM((_HB,), jnp.int32),
            pltpu.VMEM((_HB, _D), jnp.float32),
            pltpu.VMEM((_HB, _D), jnp.float32),
            pltpu.VMEM((_D // 8, 2, 8, 128), jnp.float32),
            pltpu.VMEM((_D // 8, 2, 8, 128), jnp.float32),
            pltpu.SemaphoreType.DMA,
            pltpu.SemaphoreType.DMA,
            pltpu.SemaphoreType.DMA,
            pltpu.SemaphoreType.DMA,
            pltpu.SemaphoreType.DMA,
            pltpu.SemaphoreType.DMA,
        ],
        compiler_params=pltpu.CompilerParams(
            use_tc_tiling_on_sc=False, needs_layout_passes=False,
            disable_bounds_checks=True),
    )
    return run


def kernel(x, embeddings):
    run = _build()
    out5 = run(x.T.astype(jnp.int32), embeddings)
    return out5.transpose(2, 4, 0, 1, 3).reshape(_BATCH, _HIST, _D)
